# baseline (device time: 29834 ns/iter reference)
import jax
import jax.numpy as jnp
from jax import lax
from jax.experimental import pallas as pl
from jax.experimental.pallas import tpu as pltpu

N_DEV = 8
B, Sq, D = 2, 256, 768
Hq, Hkv, Dh = 8, 2, 64
G = Hq // Hkv
SCALE = 0.125
ROWS = B * G * Sq
PARTS = ((0, 688), (688, 1376), (1376, ROWS))
SCHED = ((1, 3, 4), (3, 4, 1), (4, 1, 3))
L_SCHED = (4, 1, 3)


def kernel(x, Wq, Wo, K_ext, V_ext):
    Skv = K_ext.shape[1]
    x2 = x.reshape(B * Sq, D)

    def body(x_ref, wq_ref, wo_hbm, k_hbm, v_hbm, out_ref,
             send_buf, recv_buf, l_send, l_recv, k_vmem, v_vmem, wo_vmem,
             send_sems, recv_sems, l_send_sems, l_recv_sems,
             k_dma_sems, v_dma_sems, wo_dma_sem):
        my = lax.axis_index("i")

        dmas = {}
        for b in range(B):
            for g in range(Hkv):
                kd = pltpu.make_async_copy(
                    k_hbm.at[b, :, g, :], k_vmem.at[b, g], k_dma_sems.at[b, g])
                vd = pltpu.make_async_copy(
                    v_hbm.at[b, :, g, :], v_vmem.at[b, g], v_dma_sems.at[b, g])
                kd.start()
                vd.start()
                dmas[(b, g)] = (kd, vd)
        wo_dma = pltpu.make_async_copy(wo_hbm, wo_vmem, wo_dma_sem)
        wo_dma.start()

        q = jnp.dot(x_ref[...], wq_ref[...],
                    preferred_element_type=jnp.float32)
        ones_row = jnp.ones((1, Skv), jnp.float32)

        def partial(b, g):
            kd, vd = dmas[(b, g)]
            kd.wait()
            vd.wait()
            qg = jnp.concatenate(
                [q[b * Sq:(b + 1) * Sq, (g * G + j) * Dh:(g * G + j + 1) * Dh]
                 for j in range(G)], axis=0)
            s = lax.dot_general(
                qg, k_vmem[b, g], (((1,), (1,)), ((), ())),
                preferred_element_type=jnp.float32) * SCALE
            p = jnp.exp(s)
            acc = jnp.dot(p, v_vmem[b, g],
                          preferred_element_type=jnp.float32)
            send_buf[b * G * Sq:(b + 1) * G * Sq, g * Dh:(g + 1) * Dh] = (
                acc.astype(jnp.bfloat16))
            bg = b * Hkv + g
            l_send[bg:bg + 1, :] = lax.dot_general(
                ones_row, p, (((1,), (1,)), ((), ())),
                preferred_element_type=jnp.float32)

        def mk_rdma(ph, part):
            r0, r1 = PARTS[part]
            return pltpu.make_async_remote_copy(
                src_ref=send_buf.at[pl.ds(r0, r1 - r0), :],
                dst_ref=recv_buf.at[ph, pl.ds(r0, r1 - r0), :],
                send_sem=send_sems.at[ph, part],
                recv_sem=recv_sems.at[ph, part],
                device_id=(my ^ SCHED[part][ph],),
                device_id_type=pl.DeviceIdType.MESH)

        def mk_l_rdma(ph):
            return pltpu.make_async_remote_copy(
                src_ref=l_send, dst_ref=l_recv.at[ph],
                send_sem=l_send_sems.at[ph], recv_sem=l_recv_sems.at[ph],
                device_id=(my ^ L_SCHED[ph],),
                device_id_type=pl.DeviceIdType.MESH)

        partial(0, 0)
        partial(0, 1)
        barrier = pltpu.get_barrier_semaphore()
        for mask in (1, 3, 4):
            pl.semaphore_signal(barrier, inc=1, device_id=(my ^ mask,),
                                device_id_type=pl.DeviceIdType.MESH)
        pl.semaphore_wait(barrier, 3)

        rdma0 = mk_rdma(0, 0)
        rdma0.start()
        partial(1, 0)
        partial(1, 1)
        chains = [rdma0, mk_rdma(0, 1), mk_rdma(0, 2), mk_l_rdma(0)]
        for r in chains[1:]:
            r.start()

        def merge(ph, part):
            r0, r1 = PARTS[part]
            send_buf[r0:r1, :] = send_buf[r0:r1, :] + recv_buf[ph, r0:r1, :]

        def l_merge(ph):
            l_send[...] = l_send[...] + l_recv[ph]

        for ph in range(2):
            nxt = []
            for part in range(3):
                chains[part].wait()
                merge(ph, part)
                r = mk_rdma(ph + 1, part)
                r.start()
                nxt.append(r)
            chains[3].wait()
            l_merge(ph)
            r = mk_l_rdma(ph + 1)
            r.start()
            nxt.append(r)
            chains = nxt

        n = G * Sq
        ri = lax.broadcasted_iota(jnp.int32, (n, n), 0)
        ci = lax.broadcasted_iota(jnp.int32, (n, n), 1)
        eye = jnp.where(ri == ci, 1.0, 0.0).astype(jnp.float32)

        for part in range(3):
            chains[part].wait()
            merge(2, part)
        chains[3].wait()
        l_merge(2)

        red = send_buf[...].astype(jnp.float32)
        lcol = lax.dot_general(
            eye, l_send[...], (((1,), (1,)), ((), ())),
            preferred_element_type=jnp.float32)
        wo_dma.wait()
        for b in range(B):
            cols = []
            for hq in range(Hq):
                g, j = hq // G, hq % G
                r0 = b * n + j * Sq
                bg = b * Hkv + g
                o = (red[r0:r0 + Sq, g * Dh:(g + 1) * Dh]
                     / lcol[j * Sq:(j + 1) * Sq, bg:bg + 1])
                cols.append(o)
            row = jnp.concatenate(cols, axis=1)
            out_ref[b * Sq:(b + 1) * Sq, :] = jnp.dot(
                row, wo_vmem[...], preferred_element_type=jnp.float32)

    out = pl.pallas_call(
        body,
        out_shape=jax.ShapeDtypeStruct((B * Sq, D), jnp.float32),
        in_specs=[
            pl.BlockSpec(memory_space=pltpu.VMEM),
            pl.BlockSpec(memory_space=pltpu.VMEM),
            pl.BlockSpec(memory_space=pl.ANY),
            pl.BlockSpec(memory_space=pl.ANY),
            pl.BlockSpec(memory_space=pl.ANY),
        ],
        out_specs=pl.BlockSpec(memory_space=pltpu.VMEM),
        scratch_shapes=[
            pltpu.VMEM((ROWS, 128), jnp.bfloat16),
            pltpu.VMEM((3, ROWS, 128), jnp.bfloat16),
            pltpu.VMEM((B * Hkv, G * Sq), jnp.float32),
            pltpu.VMEM((3, B * Hkv, G * Sq), jnp.float32),
            pltpu.VMEM((B, Hkv, 512, Dh), jnp.float32),
            pltpu.VMEM((B, Hkv, 512, Dh), jnp.float32),
            pltpu.VMEM((Hq * Dh, D), jnp.float32),
            pltpu.SemaphoreType.DMA((3, 3)),
            pltpu.SemaphoreType.DMA((3, 3)),
            pltpu.SemaphoreType.DMA((3,)),
            pltpu.SemaphoreType.DMA((3,)),
            pltpu.SemaphoreType.DMA((B, Hkv)),
            pltpu.SemaphoreType.DMA((B, Hkv)),
            pltpu.SemaphoreType.DMA(()),
        ],
        compiler_params=pltpu.CompilerParams(collective_id=0),
    )(x2, Wq, Wo, K_ext, V_ext)
    return out.reshape(B, Sq, D)
